# baseline (device time: 10198 ns/iter reference)
import jax
import jax.numpy as jnp
from jax import lax
from jax.experimental import pallas as pl
from jax.experimental.pallas import tpu as pltpu

N_DEV = 8
N_TOK = 256
D_IN = 128
D_OUT = 256
N_EXP = 16
EXP_PER_DEV = 2
ROWS = N_TOK // N_DEV

ORDER = (2, 6, 3, 5, 1, 7, 4)


def kernel(x, router_W, route_idx, expert_W):
    def body(x_ref, rw_ref, idx_ref, ew_ref, out_ref,
             chunk_ref, acc_ref, send_sems, recv_sems):
        my = lax.axis_index("i")

        def chunk_partial(t):
            rows = pl.ds(t * ROWS, ROWS)
            xs = x_ref[rows, :]
            ii0 = idx_ref[rows, 0:1]
            ii1 = idx_ref[rows, 1:2]
            scores = jnp.dot(xs, rw_ref[:, :],
                             preferred_element_type=jnp.float32)
            smax = jnp.max(scores, axis=1, keepdims=True)
            es = jnp.exp(scores - smax)
            eidx = lax.broadcasted_iota(jnp.int32, (ROWS, N_EXP), 1)
            p0 = jnp.sum(jnp.where(eidx == ii0, es, 0.0), axis=1,
                         keepdims=True)
            p1 = jnp.sum(jnp.where(eidx == ii1, es, 0.0), axis=1,
                         keepdims=True)
            gs = p0 + p1
            xb = xs.astype(jnp.bfloat16)
            acc = jnp.zeros((ROWS, D_OUT), jnp.float32)
            for le in range(EXP_PER_DEV):
                eg = my * EXP_PER_DEV + le
                w = (jnp.where(ii0 == eg, p0, 0.0)
                     + jnp.where(ii1 == eg, p1, 0.0)) / gs
                y = jnp.dot(xb, ew_ref[le].astype(jnp.bfloat16),
                            preferred_element_type=jnp.float32)
                acc = acc + w * y
            return acc

        rdmas = {}
        for k in ORDER:
            t = lax.rem(my + k, N_DEV)
            chunk_ref[k, :, :] = chunk_partial(t).astype(jnp.bfloat16)
            rdma = pltpu.make_async_remote_copy(
                src_ref=chunk_ref.at[k],
                dst_ref=acc_ref.at[k],
                send_sem=send_sems.at[k],
                recv_sem=recv_sems.at[k],
                device_id=(t,),
                device_id_type=pl.DeviceIdType.MESH,
            )
            rdma.start()
            rdmas[k] = rdma

        out = chunk_partial(my)
        for k in ORDER:
            rdmas[k].wait_recv()
        for k in range(1, N_DEV):
            out = out + acc_ref[k].astype(jnp.float32)
        out_ref[:, :] = out
        for k in ORDER:
            rdmas[k].wait_send()

        bar = pltpu.get_barrier_semaphore()
        pl.semaphore_signal(bar, inc=1)
        pl.semaphore_wait(bar, 1)

    return pl.pallas_call(
        body,
        out_shape=jax.ShapeDtypeStruct((ROWS, D_OUT), jnp.float32),
        in_specs=[pl.BlockSpec(memory_space=pltpu.VMEM)] * 4,
        out_specs=pl.BlockSpec(memory_space=pltpu.VMEM),
        scratch_shapes=[
            pltpu.VMEM((N_DEV, ROWS, D_OUT), jnp.bfloat16),
            pltpu.VMEM((N_DEV, ROWS, D_OUT), jnp.bfloat16),
            pltpu.SemaphoreType.DMA((N_DEV,)),
            pltpu.SemaphoreType.DMA((N_DEV,)),
        ],
        compiler_params=pltpu.CompilerParams(collective_id=0),
    )(x, router_W, route_idx, expert_W)


# device time: 9313 ns/iter; 1.0950x vs baseline; 1.0950x over previous
import jax
import jax.numpy as jnp
from jax import lax
from jax.experimental import pallas as pl
from jax.experimental.pallas import tpu as pltpu

N_DEV = 8
N_TOK = 256
D_IN = 128
D_OUT = 256
N_EXP = 16
EXP_PER_DEV = 2
ROWS = N_TOK // N_DEV


def kernel(x, router_W, route_idx, expert_W):
    def body(x_ref, rw_ref, idx_ref, ew_ref, out_ref,
             partial_ref, diag_ref, acc_ref,
             send_sems, recv_sems):
        my = lax.axis_index("i")
        td = my ^ 6

        def gate_weights(xs, ii0, ii1, n):
            scores = jnp.dot(xs, rw_ref[:, :],
                             preferred_element_type=jnp.float32)
            smax = jnp.max(scores, axis=1, keepdims=True)
            es = jnp.exp(scores - smax)
            eidx = lax.broadcasted_iota(jnp.int32, (n, N_EXP), 1)
            p0 = jnp.sum(jnp.where(eidx == ii0, es, 0.0), axis=1,
                         keepdims=True)
            p1 = jnp.sum(jnp.where(eidx == ii1, es, 0.0), axis=1,
                         keepdims=True)
            gs = p0 + p1
            ws = []
            for le in range(EXP_PER_DEV):
                eg = my * EXP_PER_DEV + le
                ws.append((jnp.where(ii0 == eg, p0, 0.0)
                           + jnp.where(ii1 == eg, p1, 0.0)) / gs)
            return ws

        drows = pl.ds(td * ROWS, ROWS)
        xdf = x_ref[drows, :]
        wd = gate_weights(xdf, idx_ref[drows, 0:1], idx_ref[drows, 1:2], ROWS)
        xd = xdf.astype(jnp.bfloat16)
        pd = jnp.zeros((ROWS, D_OUT), jnp.float32)
        for le in range(EXP_PER_DEV):
            yd = jnp.dot(xd, ew_ref[le].astype(jnp.bfloat16),
                         preferred_element_type=jnp.float32)
            pd = pd + wd[le] * yd
        diag_ref[:, :] = pd.astype(jnp.bfloat16)

        def make_rdma(k, src):
            t = lax.rem(my + k, N_DEV)
            return pltpu.make_async_remote_copy(
                src_ref=src,
                dst_ref=acc_ref.at[k],
                send_sem=send_sems.at[k],
                recv_sem=recv_sems.at[k],
                device_id=(t,),
                device_id_type=pl.DeviceIdType.MESH,
            )

        for k in range(1, N_DEV):
            t = lax.rem(my + k, N_DEV)

            @pl.when(t == td)
            def _(k=k):
                make_rdma(k, diag_ref).start()

        xf = x_ref[:, :]
        wf = gate_weights(xf, idx_ref[:, 0:1], idx_ref[:, 1:2], N_TOK)
        xb = xf.astype(jnp.bfloat16)
        partial = jnp.zeros((N_TOK, D_OUT), jnp.float32)
        for le in range(EXP_PER_DEV):
            y = jnp.dot(xb, ew_ref[le].astype(jnp.bfloat16),
                        preferred_element_type=jnp.float32)
            partial = partial + wf[le] * y
        partial_ref[:, :] = partial.astype(jnp.bfloat16)

        waiters = {}
        for k in range(1, N_DEV):
            t = lax.rem(my + k, N_DEV)
            rdma = make_rdma(k, partial_ref.at[pl.ds(t * ROWS, ROWS)])

            @pl.when(t != td)
            def _(k=k, rdma=rdma):
                rdma.start()

            waiters[k] = rdma

        out = partial_ref[pl.ds(my * ROWS, ROWS), :].astype(jnp.float32)
        for k in (4, 1, 3, 5, 7, 2, 6):
            waiters[k].wait_recv()
            out = out + acc_ref[k].astype(jnp.float32)
        out_ref[:, :] = out
        for k in range(1, N_DEV):
            waiters[k].wait_send()

        bar = pltpu.get_barrier_semaphore()
        pl.semaphore_signal(bar, inc=1)
        pl.semaphore_wait(bar, 1)

    return pl.pallas_call(
        body,
        out_shape=jax.ShapeDtypeStruct((ROWS, D_OUT), jnp.float32),
        in_specs=[pl.BlockSpec(memory_space=pltpu.VMEM)] * 4,
        out_specs=pl.BlockSpec(memory_space=pltpu.VMEM),
        scratch_shapes=[
            pltpu.VMEM((N_TOK, D_OUT), jnp.bfloat16),
            pltpu.VMEM((ROWS, D_OUT), jnp.bfloat16),
            pltpu.VMEM((N_DEV, ROWS, D_OUT), jnp.bfloat16),
            pltpu.SemaphoreType.DMA((N_DEV,)),
            pltpu.SemaphoreType.DMA((N_DEV,)),
        ],
        compiler_params=pltpu.CompilerParams(collective_id=0),
    )(x, router_W, route_idx, expert_W)
